# baseline (speedup 1.0000x reference)
import functools

import jax
import jax.numpy as jnp
from jax import lax
from jax.experimental import pallas as pl
from jax.experimental.pallas import tpu as pltpu
from jax.experimental.pallas import tpu_sc as plsc

K = 1024
D = 1024
DDEC = 256
HALF = DDEC // 2
B = 16384

NC, NS = 2, 16
NW = NC * NS
B_PER_W = B // NW
CH = 128
NCH = B_PER_W // CH
L = 16


def _table_body(emb_ref, w_ref, b_ref, out_ref):
    acc = jnp.dot(emb_ref[...].astype(jnp.bfloat16),
                  w_ref[...].astype(jnp.bfloat16),
                  preferred_element_type=jnp.float32)
    t = jnp.tanh(acc + b_ref[...])
    lo = lax.bitcast_convert_type(t[:, :HALF].astype(jnp.bfloat16),
                                  jnp.uint16).astype(jnp.uint32)
    hi = lax.bitcast_convert_type(t[:, HALF:].astype(jnp.bfloat16),
                                  jnp.uint16).astype(jnp.uint32)
    out_ref[...] = lax.bitcast_convert_type(lo | (hi << 16), jnp.int32)


def _build_table(embedding, W_dec, b_dec):
    grid = 4
    return pl.pallas_call(
        _table_body,
        grid=(grid,),
        in_specs=[
            pl.BlockSpec((K // grid, D), lambda i: (i, 0)),
            pl.BlockSpec((D, DDEC), lambda i: (0, 0)),
            pl.BlockSpec((1, DDEC), lambda i: (0, 0)),
        ],
        out_specs=pl.BlockSpec((K // grid, HALF), lambda i: (i, 0)),
        out_shape=jax.ShapeDtypeStruct((K, HALF), jnp.int32),
    )(embedding, W_dec, b_dec.reshape(1, DDEC))


CHUNKS = (32, 96, 128, 128, 96, 32)
OFFS = tuple(sum(CHUNKS[:i]) for i in range(len(CHUNKS)))
NCHUNK = len(CHUNKS)


def _gather_body(table_hbm, idx_hbm, out_hbm, idx_v, gb, fb, gsem, osem):
    wid = lax.axis_index("s") * NC + lax.axis_index("c")
    base = wid * B_PER_W
    pltpu.sync_copy(idx_hbm.at[pl.ds(base, B_PER_W)], idx_v)

    NGB = len(gb)

    def idx_slice(ci):
        return idx_v.at[pl.ds(OFFS[ci], CHUNKS[ci])]

    def fire_gather(ci, buf):
        pltpu.async_copy(
            table_hbm.at[idx_slice(ci)],
            gb[buf].at[pl.ds(0, CHUNKS[ci])], gsem[buf])

    def wait_gather(ci, buf):
        pltpu.make_async_copy(
            table_hbm.at[idx_slice(ci)],
            gb[buf].at[pl.ds(0, CHUNKS[ci])], gsem[buf]).wait()

    def fire_store(ci, buf):
        pltpu.async_copy(fb[buf].at[pl.ds(0, CHUNKS[ci])],
                         out_hbm.at[pl.ds(base + OFFS[ci], CHUNKS[ci])],
                         osem[buf])

    def wait_store(ci, buf):
        pltpu.make_async_copy(fb[buf].at[pl.ds(0, CHUNKS[ci])],
                              out_hbm.at[pl.ds(base + OFFS[ci], CHUNKS[ci])],
                              osem[buf]).wait()

    def convert(ci, gbuf, fbuf):
        g, f = gb[gbuf], fb[fbuf]

        @plsc.parallel_loop(0, CHUNKS[ci], 1, unroll=8)
        def row(r):
            for k in range(HALF // L):
                w = lax.bitcast_convert_type(g[r, pl.ds(k * L, L)],
                                             jnp.uint32)
                f[r, pl.ds(k * L, L)] = lax.bitcast_convert_type(
                    w << 16, jnp.float32)
                f[r, pl.ds(HALF + k * L, L)] = lax.bitcast_convert_type(
                    w & jnp.uint32(0xFFFF0000), jnp.float32)

    for ci in range(min(NGB, NCHUNK)):
        fire_gather(ci, ci % NGB)
    for ci in range(NCHUNK):
        g, f = ci % NGB, ci % 2
        wait_gather(ci, g)
        if ci >= 2:
            wait_store(ci - 2, f)
        convert(ci, g, f)
        fire_store(ci, f)
        if ci + NGB < NCHUNK:
            fire_gather(ci + NGB, g)
    for ci in range(NCHUNK - 2, NCHUNK):
        wait_store(ci, ci % 2)


@functools.partial(
    pl.kernel,
    mesh=plsc.VectorSubcoreMesh(core_axis_name="c", subcore_axis_name="s"),
    out_type=jax.ShapeDtypeStruct((B, DDEC), jnp.float32),
    scratch_types=[
        pltpu.VMEM((B_PER_W,), jnp.int32),
        pltpu.VMEM((CH, HALF), jnp.int32),
        pltpu.VMEM((CH, HALF), jnp.int32),
        pltpu.VMEM((CH, HALF), jnp.int32),
        pltpu.VMEM((CH, DDEC), jnp.float32),
        pltpu.VMEM((CH, DDEC), jnp.float32),
        pltpu.SemaphoreType.DMA,
        pltpu.SemaphoreType.DMA,
        pltpu.SemaphoreType.DMA,
        pltpu.SemaphoreType.DMA,
        pltpu.SemaphoreType.DMA,
    ],
)
def _gather_rows(table_hbm, idx_hbm, out_hbm, idx_v, gb0, gb1, gb2, fb0, fb1,
                 g0, g1, g2, o0, o1):
    _gather_body(table_hbm, idx_hbm, out_hbm, idx_v,
                 (gb0, gb1, gb2), (fb0, fb1), (g0, g1, g2), (o0, o1))


def kernel(encoding_indices, embedding, W_dec, b_dec):
    bs, m = encoding_indices.shape
    table = _build_table(embedding, W_dec, b_dec)
    flat_idx = encoding_indices.reshape(-1)
    out = _gather_rows(table, flat_idx)
    return out.reshape(bs, m, DDEC)


# speedup vs baseline: 1.0177x; 1.0177x over previous
import functools

import jax
import jax.numpy as jnp
from jax import lax
from jax.experimental import pallas as pl
from jax.experimental.pallas import tpu as pltpu
from jax.experimental.pallas import tpu_sc as plsc

K = 1024
D = 1024
DDEC = 256
HALF = DDEC // 2
B = 16384

NC, NS = 2, 16
NW = NC * NS
B_PER_W = B // NW
CH = 128
NCH = B_PER_W // CH
L = 16


def _table_body(emb_ref, w_ref, b_ref, out_ref):
    acc = jnp.dot(emb_ref[...].astype(jnp.bfloat16),
                  w_ref[...].astype(jnp.bfloat16),
                  preferred_element_type=jnp.float32)
    t = jnp.tanh(acc + b_ref[...])
    lo = lax.bitcast_convert_type(t[:, :HALF].astype(jnp.bfloat16),
                                  jnp.uint16).astype(jnp.uint32)
    hi = lax.bitcast_convert_type(t[:, HALF:].astype(jnp.bfloat16),
                                  jnp.uint16).astype(jnp.uint32)
    out_ref[...] = lax.bitcast_convert_type(lo | (hi << 16), jnp.int32)


def _build_table(embedding, W_dec, b_dec):
    return pl.pallas_call(
        _table_body,
        out_shape=jax.ShapeDtypeStruct((K, HALF), jnp.int32),
    )(embedding, W_dec, b_dec.reshape(1, DDEC))


CHUNKS = (32, 96, 128, 128, 96, 32)
OFFS = tuple(sum(CHUNKS[:i]) for i in range(len(CHUNKS)))
NCHUNK = len(CHUNKS)


def _gather_body(table_hbm, idx_hbm, out_hbm, idx_v, gb, fb, gsem, osem):
    wid = lax.axis_index("s") * NC + lax.axis_index("c")
    base = wid * B_PER_W
    pltpu.sync_copy(idx_hbm.at[pl.ds(base, B_PER_W)], idx_v)

    NGB = len(gb)

    def idx_slice(ci):
        return idx_v.at[pl.ds(OFFS[ci], CHUNKS[ci])]

    def fire_gather(ci, buf):
        pltpu.async_copy(
            table_hbm.at[idx_slice(ci)],
            gb[buf].at[pl.ds(0, CHUNKS[ci])], gsem[buf])

    def wait_gather(ci, buf):
        pltpu.make_async_copy(
            table_hbm.at[idx_slice(ci)],
            gb[buf].at[pl.ds(0, CHUNKS[ci])], gsem[buf]).wait()

    def fire_store(ci, buf):
        pltpu.async_copy(fb[buf].at[pl.ds(0, CHUNKS[ci])],
                         out_hbm.at[pl.ds(base + OFFS[ci], CHUNKS[ci])],
                         osem[buf])

    def wait_store(ci, buf):
        pltpu.make_async_copy(fb[buf].at[pl.ds(0, CHUNKS[ci])],
                              out_hbm.at[pl.ds(base + OFFS[ci], CHUNKS[ci])],
                              osem[buf]).wait()

    def convert(ci, gbuf, fbuf):
        g, f = gb[gbuf], fb[fbuf]

        @plsc.parallel_loop(0, CHUNKS[ci], 1, unroll=8)
        def row(r):
            for k in range(HALF // L):
                w = lax.bitcast_convert_type(g[r, pl.ds(k * L, L)],
                                             jnp.uint32)
                f[r, pl.ds(k * L, L)] = lax.bitcast_convert_type(
                    w << 16, jnp.float32)
                f[r, pl.ds(HALF + k * L, L)] = lax.bitcast_convert_type(
                    w & jnp.uint32(0xFFFF0000), jnp.float32)

    for ci in range(min(NGB, NCHUNK)):
        fire_gather(ci, ci % NGB)
    for ci in range(NCHUNK):
        g, f = ci % NGB, ci % 2
        wait_gather(ci, g)
        if ci >= 2:
            wait_store(ci - 2, f)
        convert(ci, g, f)
        fire_store(ci, f)
        if ci + NGB < NCHUNK:
            fire_gather(ci + NGB, g)
    for ci in range(NCHUNK - 2, NCHUNK):
        wait_store(ci, ci % 2)


@functools.partial(
    pl.kernel,
    mesh=plsc.VectorSubcoreMesh(core_axis_name="c", subcore_axis_name="s"),
    out_type=jax.ShapeDtypeStruct((B, DDEC), jnp.float32),
    scratch_types=[
        pltpu.VMEM((B_PER_W,), jnp.int32),
        pltpu.VMEM((CH, HALF), jnp.int32),
        pltpu.VMEM((CH, HALF), jnp.int32),
        pltpu.VMEM((CH, HALF), jnp.int32),
        pltpu.VMEM((CH, DDEC), jnp.float32),
        pltpu.VMEM((CH, DDEC), jnp.float32),
        pltpu.SemaphoreType.DMA,
        pltpu.SemaphoreType.DMA,
        pltpu.SemaphoreType.DMA,
        pltpu.SemaphoreType.DMA,
        pltpu.SemaphoreType.DMA,
    ],
)
def _gather_rows(table_hbm, idx_hbm, out_hbm, idx_v, gb0, gb1, gb2, fb0, fb1,
                 g0, g1, g2, o0, o1):
    _gather_body(table_hbm, idx_hbm, out_hbm, idx_v,
                 (gb0, gb1, gb2), (fb0, fb1), (g0, g1, g2), (o0, o1))


def kernel(encoding_indices, embedding, W_dec, b_dec):
    bs, m = encoding_indices.shape
    table = _build_table(embedding, W_dec, b_dec)
    flat_idx = encoding_indices.reshape(-1)
    out = _gather_rows(table, flat_idx)
    return out.reshape(bs, m, DDEC)


# speedup vs baseline: 1.0224x; 1.0046x over previous
import functools

import jax
import jax.numpy as jnp
from jax import lax
from jax.experimental import pallas as pl
from jax.experimental.pallas import tpu as pltpu
from jax.experimental.pallas import tpu_sc as plsc

K = 1024
D = 1024
DDEC = 256
HALF = DDEC // 2
B = 16384

NC, NS = 2, 16
NW = NC * NS
B_PER_W = B // NW
CH = 128
L = 16


def _table_body(emb_ref, w_ref, b_ref, out_ref):
    acc = jnp.dot(emb_ref[...].astype(jnp.bfloat16),
                  w_ref[...].astype(jnp.bfloat16),
                  preferred_element_type=jnp.float32)
    t = jnp.tanh(acc + b_ref[...])
    lo = lax.bitcast_convert_type(t[:, :HALF].astype(jnp.bfloat16),
                                  jnp.uint16).astype(jnp.uint32)
    hi = lax.bitcast_convert_type(t[:, HALF:].astype(jnp.bfloat16),
                                  jnp.uint16).astype(jnp.uint32)
    out_ref[...] = lax.bitcast_convert_type(lo | (hi << 16), jnp.int32)


def _build_table(embedding, W_dec, b_dec):
    return pl.pallas_call(
        _table_body,
        out_shape=jax.ShapeDtypeStruct((K, HALF), jnp.int32),
    )(embedding, W_dec, b_dec.reshape(1, DDEC))


CHUNKS = (32, 96, 128, 128, 96, 32)
OFFS = tuple(sum(CHUNKS[:i]) for i in range(len(CHUNKS)))
NCHUNK = len(CHUNKS)


def _gather_body(table_hbm, idx_hbm, out_hbm, idx_v, gb, fb, gsem, osem):
    wid = lax.axis_index("s") * NC + lax.axis_index("c")
    base = wid * B_PER_W
    pltpu.sync_copy(idx_hbm.at[pl.ds(base, B_PER_W)], idx_v)

    NGB = len(gb)

    def idx_slice(ci):
        return idx_v.at[pl.ds(OFFS[ci], CHUNKS[ci])]

    def fire_gather(ci, buf):
        pltpu.async_copy(
            table_hbm.at[idx_slice(ci)],
            gb[buf].at[pl.ds(0, CHUNKS[ci])], gsem[buf])

    def wait_gather(ci, buf):
        pltpu.make_async_copy(
            table_hbm.at[idx_slice(ci)],
            gb[buf].at[pl.ds(0, CHUNKS[ci])], gsem[buf]).wait()

    def fire_store(ci, buf):
        pltpu.async_copy(fb[buf].at[pl.ds(0, CHUNKS[ci])],
                         out_hbm.at[pl.ds(base + OFFS[ci], CHUNKS[ci])],
                         osem[buf])

    def wait_store(ci, buf):
        pltpu.make_async_copy(fb[buf].at[pl.ds(0, CHUNKS[ci])],
                              out_hbm.at[pl.ds(base + OFFS[ci], CHUNKS[ci])],
                              osem[buf]).wait()

    def convert(ci, gbuf, fbuf):
        g, f = gb[gbuf], fb[fbuf]

        @plsc.parallel_loop(0, CHUNKS[ci], 1, unroll=8)
        def row(r):
            for k in range(HALF // L):
                w = lax.bitcast_convert_type(g[r, pl.ds(k * L, L)],
                                             jnp.uint32)
                f[r, pl.ds(k * L, L)] = lax.bitcast_convert_type(
                    w << 16, jnp.float32)
                f[r, pl.ds(HALF + k * L, L)] = lax.bitcast_convert_type(
                    w & jnp.uint32(0xFFFF0000), jnp.float32)

    for ci in range(min(NGB, NCHUNK)):
        fire_gather(ci, ci % NGB)
    for ci in range(NCHUNK):
        g, f = ci % NGB, ci % 2
        wait_gather(ci, g)
        if ci >= 2:
            wait_store(ci - 2, f)
        convert(ci, g, f)
        fire_store(ci, f)
        if ci + NGB < NCHUNK:
            fire_gather(ci + NGB, g)
    for ci in range(NCHUNK - 2, NCHUNK):
        wait_store(ci, ci % 2)


@functools.partial(
    pl.kernel,
    mesh=plsc.VectorSubcoreMesh(core_axis_name="c", subcore_axis_name="s"),
    out_type=jax.ShapeDtypeStruct((B, DDEC), jnp.float32),
    scratch_types=[
        pltpu.VMEM((B_PER_W,), jnp.int32),
        pltpu.VMEM((CH, HALF), jnp.int32),
        pltpu.VMEM((CH, HALF), jnp.int32),
        pltpu.VMEM((CH, HALF), jnp.int32),
        pltpu.VMEM((CH, DDEC), jnp.float32),
        pltpu.VMEM((CH, DDEC), jnp.float32),
        pltpu.SemaphoreType.DMA,
        pltpu.SemaphoreType.DMA,
        pltpu.SemaphoreType.DMA,
        pltpu.SemaphoreType.DMA,
        pltpu.SemaphoreType.DMA,
    ],
)
def _gather_rows(table_hbm, idx_hbm, out_hbm, idx_v, gb0, gb1, gb2, fb0, fb1,
                 g0, g1, g2, o0, o1):
    _gather_body(table_hbm, idx_hbm, out_hbm, idx_v,
                 (gb0, gb1, gb2), (fb0, fb1), (g0, g1, g2), (o0, o1))


def kernel(encoding_indices, embedding, W_dec, b_dec):
    bs, m = encoding_indices.shape
    table = _build_table(embedding, W_dec, b_dec)
    flat_idx = encoding_indices.reshape(-1)
    out = _gather_rows(table, flat_idx)
    return out.reshape(bs, m, DDEC)
